# baseline (speedup 1.0000x reference)
import jax
import jax.numpy as jnp
from jax.experimental import pallas as pl
from jax.experimental.pallas import tpu as pltpu

_B, _C = 16384, 1000
_K = 12288
_ROWS = 1024
_NBLK = _B // _ROWS

_INT_MIN = -2147483648


def _loss_body(x_ref, t_ref, loss_ref):
    x = x_ref[...]
    t = t_ref[0, 0, :]
    m = jnp.max(x, axis=1)
    s = jnp.sum(jnp.exp(x - m[:, None]), axis=1)
    cols = jax.lax.broadcasted_iota(jnp.int32, (_ROWS, _C), 1)
    picked = jnp.sum(jnp.where(cols == t[:, None], x, 0.0), axis=1)
    loss_ref[0, 0, :] = jnp.log(s) + m - picked


def _topk_body(loss_ref, out_ref):
    x = loss_ref[...]
    bits = jax.lax.bitcast_convert_type(x, jnp.int32)
    key = jnp.where(bits >= 0, bits, bits ^ jnp.int32(0x7FFFFFFF))

    def body(i, T):
        cand = T | jax.lax.shift_left(jnp.int32(1), jnp.int32(31) - i)
        scand = cand ^ jnp.int32(_INT_MIN)
        cnt = jnp.sum((key >= scand).astype(jnp.int32))
        return jnp.where(cnt >= _K, cand, T)

    T = jax.lax.fori_loop(0, 32, body, jnp.int32(0))
    kth = T ^ jnp.int32(_INT_MIN)

    gt = key > kth
    cnt_gt = jnp.sum(gt.astype(jnp.int32))
    sum_gt = jnp.sum(jnp.where(gt, x, 0.0))
    tval = jnp.max(jnp.where(key == kth, x, -jnp.inf))
    res = (sum_gt + (_K - cnt_gt).astype(jnp.float32) * tval) / _K
    out_ref[...] = jnp.full((1, 1), res, jnp.float32)


def kernel(input, target):
    t3 = target.reshape(_NBLK, 1, _ROWS)
    loss = pl.pallas_call(
        _loss_body,
        grid=(_NBLK,),
        in_specs=[
            pl.BlockSpec((_ROWS, _C), lambda i: (i, 0)),
            pl.BlockSpec((1, 1, _ROWS), lambda i: (i, 0, 0)),
        ],
        out_specs=pl.BlockSpec((1, 1, _ROWS), lambda i: (i, 0, 0)),
        out_shape=jax.ShapeDtypeStruct((_NBLK, 1, _ROWS), jnp.float32),
    )(input, t3)

    loss2d = loss.reshape(128, 128)
    out = pl.pallas_call(
        _topk_body,
        out_shape=jax.ShapeDtypeStruct((1, 1), jnp.float32),
    )(loss2d)
    return out[0, 0]


# speedup vs baseline: 2.6518x; 2.6518x over previous
import jax
import jax.numpy as jnp
from jax.experimental import pallas as pl
from jax.experimental.pallas import tpu as pltpu

_B, _C = 16384, 1000
_K = 12288
_COLS = 1024
_NBLK = _B // _COLS

_INT_MIN = -2147483648


def _loss_body(x_ref, t_ref, loss_ref):
    x = x_ref[...]
    t = t_ref[...]
    m = jnp.max(x, axis=0)
    s = jnp.sum(jnp.exp(x - m[None, :]), axis=0)
    rows = jax.lax.broadcasted_iota(jnp.int32, (_C, _COLS), 0)
    picked = jnp.sum(jnp.where(rows == t, x, 0.0), axis=0)
    loss_ref[...] = (jnp.log(s) + m - picked)[None, :]


def _topk_body(loss_ref, out_ref):
    x = loss_ref[...]
    bits = jax.lax.bitcast_convert_type(x, jnp.int32)
    key = jnp.where(bits >= 0, bits, bits ^ jnp.int32(0x7FFFFFFF))

    def body(i, T):
        cand = T | jax.lax.shift_left(jnp.int32(1), jnp.int32(31) - i)
        scand = cand ^ jnp.int32(_INT_MIN)
        cnt = jnp.sum((key >= scand).astype(jnp.int32))
        return jnp.where(cnt >= _K, cand, T)

    T = jax.lax.fori_loop(0, 32, body, jnp.int32(0))
    kth = T ^ jnp.int32(_INT_MIN)

    gt = key > kth
    cnt_gt = jnp.sum(gt.astype(jnp.int32))
    sum_gt = jnp.sum(jnp.where(gt, x, 0.0))
    tval = jnp.max(jnp.where(key == kth, x, -jnp.inf))
    res = (sum_gt + (_K - cnt_gt).astype(jnp.float32) * tval) / _K
    out_ref[...] = jnp.full((1, 1), res, jnp.float32)


def kernel(input, target):
    xt = input.T
    t2 = target.reshape(1, _B)
    loss = pl.pallas_call(
        _loss_body,
        grid=(_NBLK,),
        in_specs=[
            pl.BlockSpec((_C, _COLS), lambda i: (0, i)),
            pl.BlockSpec((1, _COLS), lambda i: (0, i)),
        ],
        out_specs=pl.BlockSpec((1, _COLS), lambda i: (0, i)),
        out_shape=jax.ShapeDtypeStruct((1, _B), jnp.float32),
    )(xt, t2)

    loss2d = loss.reshape(128, 128)
    out = pl.pallas_call(
        _topk_body,
        out_shape=jax.ShapeDtypeStruct((1, 1), jnp.float32),
    )(loss2d)
    return out[0, 0]
